# baseline (speedup 1.0000x reference)
import functools

import jax
import jax.numpy as jnp
from jax import lax
from jax.experimental import pallas as pl
from jax.experimental.pallas import tpu as pltpu
from jax.experimental.pallas import tpu_sc as plsc

EMB = 16
BATCH = 16384


@functools.lru_cache(maxsize=None)
def _make_lookup_kernel(batch: int, emb: int):
    info = plsc.get_sparse_core_info()
    nw = info.num_cores * info.num_subcores
    bpw = batch // nw
    mesh = plsc.VectorSubcoreMesh(core_axis_name="c", subcore_axis_name="s")

    @functools.partial(
        pl.kernel,
        mesh=mesh,
        compiler_params=pltpu.CompilerParams(use_tc_tiling_on_sc=False),
        out_type=[
            jax.ShapeDtypeStruct((batch, emb), jnp.float32),
            jax.ShapeDtypeStruct((batch, emb), jnp.float32),
        ],
        scratch_types=[
            pltpu.VMEM((bpw,), jnp.int32),
            pltpu.VMEM((bpw,), jnp.int32),
            pltpu.VMEM((bpw, emb), jnp.float32),
            pltpu.VMEM((bpw, emb), jnp.float32),
            pltpu.SemaphoreType.DMA,
            pltpu.SemaphoreType.DMA,
        ],
    )
    def lookup(users_hbm, items_hbm, uemb_hbm, iemb_hbm, out_u_hbm, out_i_hbm,
               idx_u, idx_i, rows_u, rows_i, sem_u, sem_i):
        wid = lax.axis_index("s") * info.num_cores + lax.axis_index("c")
        base = wid * bpw
        pltpu.sync_copy(users_hbm.at[pl.ds(base, bpw)], idx_u)
        pltpu.sync_copy(items_hbm.at[pl.ds(base, bpw)], idx_i)
        cu = pltpu.async_copy(uemb_hbm.at[idx_u], rows_u, sem_u)
        ci = pltpu.async_copy(iemb_hbm.at[idx_i], rows_i, sem_i)
        cu.wait()
        ci.wait()
        pltpu.sync_copy(rows_u, out_u_hbm.at[pl.ds(base, bpw)])
        pltpu.sync_copy(rows_i, out_i_hbm.at[pl.ds(base, bpw)])

    return lookup


def kernel(users, items, user_emb, item_emb):
    batch, = users.shape
    emb = user_emb.shape[1]
    lookup = _make_lookup_kernel(batch, emb)
    u, i = lookup(users.astype(jnp.int32), items.astype(jnp.int32),
                  user_emb, item_emb)
    return (u, i)


# speedup vs baseline: 6.0105x; 6.0105x over previous
import functools

import jax
import jax.numpy as jnp
from jax import lax
from jax.experimental import pallas as pl
from jax.experimental.pallas import tpu as pltpu
from jax.experimental.pallas import tpu_sc as plsc

EMB = 16
BATCH = 16384
CHUNK = 16
LANE = 128


@functools.lru_cache(maxsize=None)
def _make_lookup_kernel(batch: int, emb: int):
    info = plsc.get_sparse_core_info()
    nw = info.num_cores * info.num_subcores
    bpw = batch // nw
    n_chunks = bpw // CHUNK
    mesh = plsc.VectorSubcoreMesh(core_axis_name="c", subcore_axis_name="s")

    @functools.partial(
        pl.kernel,
        mesh=mesh,
        compiler_params=pltpu.CompilerParams(needs_layout_passes=False),
        out_type=[
            jax.ShapeDtypeStruct((emb, batch), jnp.float32),
            jax.ShapeDtypeStruct((emb, batch), jnp.float32),
        ],
        scratch_types=[
            pltpu.VMEM((bpw,), jnp.int32),
            pltpu.VMEM((bpw,), jnp.int32),
            pltpu.VMEM((CHUNK, emb, LANE), jnp.float32),
            pltpu.VMEM((CHUNK, emb, LANE), jnp.float32),
            pltpu.VMEM((emb, bpw), jnp.float32),
            pltpu.VMEM((emb, bpw), jnp.float32),
            pltpu.SemaphoreType.DMA,
            pltpu.SemaphoreType.DMA,
        ],
    )
    def lookup(users_hbm, items_hbm, ut_hbm, it_hbm, ou_hbm, oi_hbm,
               idx_u, idx_i, ring0, ring1, out_u, out_i, sem0, sem1):
        wid = lax.axis_index("s") * info.num_cores + lax.axis_index("c")
        base = wid * bpw
        pltpu.sync_copy(users_hbm.at[pl.ds(base, bpw)], idx_u)
        pltpu.sync_copy(items_hbm.at[pl.ds(base, bpw)], idx_i)
        iota = lax.iota(jnp.int32, 16)

        def fetch(c, idx_ref, tab_hbm, ring, sem):
            off = pl.multiple_of(c * CHUNK, CHUNK)
            vec = idx_ref[pl.ds(off, CHUNK)]
            for lane in range(CHUNK):
                blk = pl.multiple_of((vec[lane] >> 7) * LANE, LANE)
                pltpu.async_copy(tab_hbm.at[:, pl.ds(blk, LANE)],
                                 ring.at[lane], sem)

        def extract(c, idx_ref, tab_hbm, ring, sem, out):
            off = pl.multiple_of(c * CHUNK, CHUNK)
            vec = idx_ref[pl.ds(off, CHUNK)]
            for lane in range(CHUNK):
                pltpu.make_async_copy(tab_hbm.at[:, pl.ds(0, LANE)],
                                      ring.at[lane], sem).wait()
            for lane in range(CHUNK):
                col = jnp.broadcast_to(vec[lane] & (LANE - 1), (16,))
                dst = jnp.broadcast_to(off + lane, (16,))
                val = plsc.load_gather(ring.at[lane], [iota, col])
                plsc.store_scatter(out, [iota, dst], val)

        fetch(0, idx_u, ut_hbm, ring0, sem0)

        @pl.loop(0, 2 * n_chunks - 1)
        def pipeline(vc):
            c = vc >> 1
            even = (vc & 1) == 0

            @pl.when(even)
            def _():
                fetch(c, idx_i, it_hbm, ring1, sem1)
                extract(c, idx_u, ut_hbm, ring0, sem0, out_u)

            @pl.when(jnp.logical_not(even))
            def _():
                fetch(c + 1, idx_u, ut_hbm, ring0, sem0)
                extract(c, idx_i, it_hbm, ring1, sem1, out_i)

        extract(n_chunks - 1, idx_i, it_hbm, ring1, sem1, out_i)

        pltpu.sync_copy(out_u, ou_hbm.at[:, pl.ds(base, bpw)])
        pltpu.sync_copy(out_i, oi_hbm.at[:, pl.ds(base, bpw)])

    return lookup


def kernel(users, items, user_emb, item_emb):
    batch, = users.shape
    emb = user_emb.shape[1]
    lookup = _make_lookup_kernel(batch, emb)
    ou, oi = lookup(users.astype(jnp.int32), items.astype(jnp.int32),
                    user_emb.T, item_emb.T)
    return (ou.T, oi.T)
